# baseline (speedup 1.0000x reference)
import functools
import math

import jax
import jax.numpy as jnp
from jax import lax
from jax.experimental import pallas as pl
from jax.experimental.pallas import tpu as pltpu
from jax.experimental.pallas import tpu_sc as plsc

N = 64
H = 8
CIN = 768
COUT = 768
B = 8192
LAYER_NUM = 5

BM = 256
MT = B // BM
G = MT + N - 1


def _sc_row_gather(table, idx):
    R, D = table.shape
    info = plsc.get_sparse_core_info()
    NC, NS = info.num_cores, info.num_subcores
    NW = NC * NS
    rows_per_w = R // NW
    CH = 128
    n_chunks = rows_per_w // CH

    mesh = plsc.VectorSubcoreMesh(core_axis_name="c", subcore_axis_name="s")

    @functools.partial(
        pl.kernel,
        mesh=mesh,
        out_type=jax.ShapeDtypeStruct((R, D), table.dtype),
        scratch_types=[
            pltpu.VMEM((CH,), jnp.int32),
            pltpu.VMEM((CH, D), table.dtype),
            pltpu.SemaphoreType.DMA,
        ],
    )
    def k(table_hbm, idx_hbm, out_hbm, idx_v, rows_v, sem):
        wid = lax.axis_index("s") * NC + lax.axis_index("c")
        for c in range(n_chunks):
            base = wid * rows_per_w + c * CH
            pltpu.sync_copy(idx_hbm.at[pl.ds(base, CH)], idx_v)
            pltpu.async_copy(table_hbm.at[idx_v], rows_v, sem).wait()
            pltpu.sync_copy(rows_v, out_hbm.at[pl.ds(base, CH)])

    return k(table, idx)


def _gmm_body(grp_s, mt_s, lo_s, hi_s, x_ref, w_ref, b_ref, o_ref):
    j = pl.program_id(0)
    mt = mt_s[j]
    lo = lo_s[j]
    hi = hi_s[j]
    rows = mt * BM + lax.broadcasted_iota(jnp.int32, (BM, 1), 0)
    mask = (rows >= lo) & (rows < hi)

    x = x_ref[...].astype(jnp.bfloat16)
    w = w_ref[0].astype(jnp.bfloat16)
    part = lax.dot_general(
        x, w, (((1,), (1,)), ((), ())), preferred_element_type=jnp.float32
    )
    part = jnp.where(mask, part, 0.0)

    prev_mt = mt_s[jnp.maximum(j - 1, 0)]
    next_mt = mt_s[jnp.minimum(j + 1, G - 1)]
    is_first = (j == 0) | (mt != prev_mt)
    is_last = (j == G - 1) | (mt != next_mt)

    prev = jnp.where(is_first, jnp.zeros_like(part), o_ref[...])
    acc = prev + part
    final = acc + b_ref[...]
    final = jnp.where(final >= 0, final, 0.2 * final)
    o_ref[...] = jnp.where(is_last, final, acc)


def _gmm(x_sorted, W, bias2d, grp, mt, lo, hi):
    grid_spec = pltpu.PrefetchScalarGridSpec(
        num_scalar_prefetch=4,
        grid=(G,),
        in_specs=[
            pl.BlockSpec((BM, CIN), lambda j, g, m, l, h: (m[j], 0)),
            pl.BlockSpec((1, COUT, CIN), lambda j, g, m, l, h: (g[j], 0, 0)),
            pl.BlockSpec((1, COUT), lambda j, g, m, l, h: (0, 0)),
        ],
        out_specs=pl.BlockSpec((BM, COUT), lambda j, g, m, l, h: (m[j], 0)),
    )
    return pl.pallas_call(
        _gmm_body,
        grid_spec=grid_spec,
        out_shape=jax.ShapeDtypeStruct((B, COUT), jnp.float32),
        compiler_params=pltpu.CompilerParams(
            dimension_semantics=("arbitrary",),
        ),
    )(grp, mt, lo, hi, x_sorted, W, bias2d)


def _routing(in_coords):
    A = 2 ** (LAYER_NUM - 1)
    b = 0.5
    aff = in_coords * A + b
    xg = jnp.floor(aff[:, 0]).astype(jnp.int32) % H
    yg = jnp.floor(aff[:, 1]).astype(jnp.int32) % H
    tile = H * xg + yg

    perm = jnp.argsort(tile).astype(jnp.int32)
    inv_perm = (
        jnp.zeros((B,), jnp.int32).at[perm].set(jnp.arange(B, dtype=jnp.int32))
    )

    sizes = jnp.bincount(tile, length=N).astype(jnp.int32)
    ends = jnp.cumsum(sizes)
    starts = ends - sizes
    blocks = jnp.where(sizes > 0, (ends - 1) // BM - starts // BM + 1, 0)
    u = jnp.cumsum(blocks)
    total = u[-1]

    j = jnp.arange(G, dtype=jnp.int32)
    g = jnp.searchsorted(u, j, side="right").astype(jnp.int32)
    valid = j < total
    gc = jnp.minimum(g, N - 1)
    k_in_g = j - (u[gc] - blocks[gc])
    mt = jnp.where(valid, starts[gc] // BM + k_in_g, MT - 1).astype(jnp.int32)
    lo = jnp.where(valid, jnp.maximum(starts[gc], mt * BM), B).astype(jnp.int32)
    hi = jnp.where(
        valid, jnp.minimum(ends[gc], (mt + 1) * BM), B
    ).astype(jnp.int32)
    grp = jnp.where(valid, gc, N - 1).astype(jnp.int32)
    return perm, inv_perm, grp, mt, lo, hi


def kernel(in_feats, in_coords, W, bias):
    perm, inv_perm, grp, mt, lo, hi = _routing(in_coords)
    x_sorted = _sc_row_gather(in_feats, perm)
    bias2d = bias.reshape(1, COUT)
    out_sorted = _gmm(x_sorted, W, bias2d, grp, mt, lo, hi)
    return _sc_row_gather(out_sorted, inv_perm)


# speedup vs baseline: 1.0266x; 1.0266x over previous
import functools

import jax
import jax.numpy as jnp
from jax import lax
from jax.experimental import pallas as pl
from jax.experimental.pallas import tpu as pltpu
from jax.experimental.pallas import tpu_sc as plsc

N = 64
H = 8
CIN = 768
COUT = 768
B = 8192
LAYER_NUM = 5

BM = 256
MT = B // BM
G = MT + N

NV = B // 16


def _sc_router(in_feats, in_coords):
    info = plsc.get_sparse_core_info()
    NC, NS = info.num_cores, info.num_subcores
    NW = NC * NS
    TPW = B // NW
    VPW = TPW // 16

    mesh = plsc.VectorSubcoreMesh(core_axis_name="c", subcore_axis_name="s")

    @functools.partial(
        pl.kernel,
        mesh=mesh,
        out_type=(
            jax.ShapeDtypeStruct((B, CIN), jnp.float32),
            jax.ShapeDtypeStruct((B,), jnp.int32),
            jax.ShapeDtypeStruct((4, G), jnp.int32),
        ),
        scratch_types=[
            pltpu.VMEM((2 * B,), jnp.float32),
            pltpu.VMEM((B,), jnp.int32),
            pltpu.VMEM((TPW,), jnp.int32),
            pltpu.VMEM((64,), jnp.int32),
            pltpu.VMEM((64, CIN), jnp.float32),
            pltpu.VMEM((16,), jnp.int32),
            pltpu.VMEM((16,), jnp.int32),
            pltpu.VMEM((N,), jnp.int32),
            pltpu.VMEM((N,), jnp.int32),
            pltpu.VMEM((N,), jnp.int32),
            pltpu.VMEM((N,), jnp.int32),
            pltpu.VMEM((N,), jnp.int32),
            pltpu.VMEM((N,), jnp.int32),
            pltpu.VMEM((G,), jnp.int32),
            pltpu.VMEM((G,), jnp.int32),
            pltpu.VMEM((G,), jnp.int32),
            pltpu.VMEM((G,), jnp.int32),
            pltpu.SemaphoreType.DMA,
        ],
        compiler_params=pltpu.CompilerParams(needs_layout_passes=False),
    )
    def k(feats_hbm, coords_hbm, xs_hbm, pos_hbm, meta_hbm,
          coords_v, tiles_v, pos_v, idx64_v, rows_v, skv, tmp16,
          cnt_all, cnt_pre, base_run, starts_v, blocks_v, u_v,
          grp_v, mt_v, lo_v, hi_v, sem):
        wid = lax.axis_index("s") * NC + lax.axis_index("c")
        iota = jnp.arange(16, dtype=jnp.int32)

        pltpu.sync_copy(coords_hbm, coords_v)

        def f2i_floor(a):
            t = a.astype(jnp.int32)
            return t - (t.astype(jnp.float32) > a).astype(jnp.int32)

        def tile_body(kv, _):
            rows2 = (kv * 16 + iota) * 2
            cx = plsc.load_gather(coords_v, [rows2])
            cy = plsc.load_gather(coords_v, [rows2 + 1])
            tx = f2i_floor(cx * 16.0 + 0.5) & (H - 1)
            ty = f2i_floor(cy * 16.0 + 0.5) & (H - 1)
            tiles_v[pl.ds(kv * 16, 16)] = tx * H + ty
            return 0

        lax.fori_loop(0, NV, tile_body, 0)

        zeros16 = jnp.zeros((16,), jnp.int32)
        for c in range(N // 16):
            cnt_all[pl.ds(c * 16, 16)] = zeros16
            cnt_pre[pl.ds(c * 16, 16)] = zeros16

        def vreg_stats(keys):
            sk, sl = plsc.sort_key_val(keys, iota)
            skv[...] = sk
            prv = plsc.load_gather(skv, [jnp.maximum(iota - 1, 0)])
            nxt = plsc.load_gather(skv, [jnp.minimum(iota + 1, 15)])
            change = (iota == 0) | (sk != prv)
            first = plsc.cummax(jnp.where(change, iota, 0))
            rank = iota - first
            is_last = (iota == 15) | (sk != nxt)
            return sk, sl, rank, is_last

        myv0 = wid * VPW

        def hist_body(kv, _):
            keys = tiles_v[pl.ds(kv * 16, 16)]
            sk, _sl, rank, is_last = vreg_stats(keys)
            cnt = rank + 1
            plsc.addupdate_scatter(cnt_all, [sk], cnt, mask=is_last)
            plsc.addupdate_scatter(
                cnt_pre, [sk], cnt, mask=is_last & (kv < myv0)
            )
            return 0

        lax.fori_loop(0, NV, hist_body, 0)

        carry = jnp.int32(0)
        for c in range(N // 16):
            tot = cnt_all[pl.ds(c * 16, 16)]
            start = plsc.cumsum(tot) - tot + carry
            carry = carry + jnp.sum(tot)
            starts_v[pl.ds(c * 16, 16)] = start
            base_run[pl.ds(c * 16, 16)] = start + cnt_pre[pl.ds(c * 16, 16)]

        def pos_body(kv, _):
            keys = tiles_v[pl.ds((myv0 + kv) * 16, 16)]
            sk, sl, rank, is_last = vreg_stats(keys)
            basek = plsc.load_gather(base_run, [sk])
            plsc.store_scatter(tmp16, [sl], basek + rank)
            pos_v[pl.ds(kv * 16, 16)] = tmp16[...]
            plsc.addupdate_scatter(base_run, [sk], rank + 1, mask=is_last)
            return 0

        lax.fori_loop(0, VPW, pos_body, 0)
        pltpu.sync_copy(pos_v, pos_hbm.at[pl.ds(wid * TPW, TPW)])

        for c in range(TPW // 64):
            b0 = wid * TPW + c * 64
            pltpu.sync_copy(feats_hbm.at[pl.ds(b0, 64)], rows_v)
            for q in range(4):
                idx64_v[pl.ds(q * 16, 16)] = pos_v[pl.ds(c * 64 + q * 16, 16)]
            pltpu.async_copy(rows_v, xs_hbm.at[idx64_v], sem).wait()

        @pl.when(wid == 0)
        def _():
            carry2 = jnp.int32(0)
            for c in range(N // 16):
                sz = cnt_all[pl.ds(c * 16, 16)]
                st = starts_v[pl.ds(c * 16, 16)]
                en = st + sz
                blk = jnp.where(
                    sz > 0,
                    ((en - 1) >> 8) - (st >> 8) + 1,
                    0,
                )
                blocks_v[pl.ds(c * 16, 16)] = blk
                u_v[pl.ds(c * 16, 16)] = plsc.cumsum(blk) + carry2
                carry2 = carry2 + jnp.sum(blk)

            total = plsc.load_gather(u_v, [jnp.full((16,), N - 1, jnp.int32)])

            def meta_body(jv, _):
                j = jv * 16 + iota
                g0 = jnp.zeros((16,), jnp.int32)

                def count_body(t, g):
                    ut = plsc.load_gather(
                        u_v, [jnp.full((16,), t, jnp.int32)]
                    )
                    return g + (ut <= j).astype(jnp.int32)

                g = lax.fori_loop(0, N, count_body, g0)
                gc = jnp.minimum(g, N - 1)
                u_gc = plsc.load_gather(u_v, [gc])
                blk_gc = plsc.load_gather(blocks_v, [gc])
                st_gc = plsc.load_gather(starts_v, [gc])
                sz_gc = plsc.load_gather(cnt_all, [gc])
                en_gc = st_gc + sz_gc
                valid = j < total
                k_in = j - (u_gc - blk_gc)
                mtj = jnp.where(valid, (st_gc >> 8) + k_in, MT - 1)
                loj = jnp.where(valid, jnp.maximum(st_gc, mtj * BM), B)
                hij = jnp.where(valid, jnp.minimum(en_gc, (mtj + 1) * BM), B)
                grpj = jnp.where(valid, gc, N - 1)
                grp_v[pl.ds(jv * 16, 16)] = grpj
                mt_v[pl.ds(jv * 16, 16)] = mtj
                lo_v[pl.ds(jv * 16, 16)] = loj
                hi_v[pl.ds(jv * 16, 16)] = hij
                return 0

            lax.fori_loop(0, G // 16, meta_body, 0)
            pltpu.sync_copy(grp_v, meta_hbm.at[0])
            pltpu.sync_copy(mt_v, meta_hbm.at[1])
            pltpu.sync_copy(lo_v, meta_hbm.at[2])
            pltpu.sync_copy(hi_v, meta_hbm.at[3])

    return k(in_feats, in_coords.reshape(2 * B))


def _sc_row_gather(table, idx):
    R, D = table.shape
    info = plsc.get_sparse_core_info()
    NC, NS = info.num_cores, info.num_subcores
    NW = NC * NS
    rows_per_w = R // NW
    CH = 128
    n_chunks = rows_per_w // CH

    mesh = plsc.VectorSubcoreMesh(core_axis_name="c", subcore_axis_name="s")

    @functools.partial(
        pl.kernel,
        mesh=mesh,
        out_type=jax.ShapeDtypeStruct((R, D), table.dtype),
        scratch_types=[
            pltpu.VMEM((CH,), jnp.int32),
            pltpu.VMEM((CH, D), table.dtype),
            pltpu.SemaphoreType.DMA,
        ],
    )
    def k(table_hbm, idx_hbm, out_hbm, idx_v, rows_v, sem):
        wid = lax.axis_index("s") * NC + lax.axis_index("c")
        for c in range(n_chunks):
            base = wid * rows_per_w + c * CH
            pltpu.sync_copy(idx_hbm.at[pl.ds(base, CH)], idx_v)
            pltpu.async_copy(table_hbm.at[idx_v], rows_v, sem).wait()
            pltpu.sync_copy(rows_v, out_hbm.at[pl.ds(base, CH)])

    return k(table, idx)


def _gmm_body(meta_s, x_ref, w_ref, b_ref, o_ref):
    j = pl.program_id(0)
    mt = meta_s[1, j]
    lo = meta_s[2, j]
    hi = meta_s[3, j]
    rows = mt * BM + lax.broadcasted_iota(jnp.int32, (BM, 1), 0)
    mask = (rows >= lo) & (rows < hi)

    x = x_ref[...].astype(jnp.bfloat16)
    w = w_ref[0].astype(jnp.bfloat16)
    part = lax.dot_general(
        x, w, (((1,), (1,)), ((), ())), preferred_element_type=jnp.float32
    )
    part = jnp.where(mask, part, 0.0)

    prev_mt = meta_s[1, jnp.maximum(j - 1, 0)]
    next_mt = meta_s[1, jnp.minimum(j + 1, G - 1)]
    is_first = (j == 0) | (mt != prev_mt)
    is_last = (j == G - 1) | (mt != next_mt)

    prev = jnp.where(is_first, jnp.zeros_like(part), o_ref[...])
    acc = prev + part
    final = acc + b_ref[...]
    final = jnp.where(final >= 0, final, 0.2 * final)
    o_ref[...] = jnp.where(is_last, final, acc)


def _gmm(x_sorted, W, bias2d, meta):
    grid_spec = pltpu.PrefetchScalarGridSpec(
        num_scalar_prefetch=1,
        grid=(G,),
        in_specs=[
            pl.BlockSpec((BM, CIN), lambda j, m: (m[1, j], 0)),
            pl.BlockSpec((1, COUT, CIN), lambda j, m: (m[0, j], 0, 0)),
            pl.BlockSpec((1, COUT), lambda j, m: (0, 0)),
        ],
        out_specs=pl.BlockSpec((BM, COUT), lambda j, m: (m[1, j], 0)),
    )
    return pl.pallas_call(
        _gmm_body,
        grid_spec=grid_spec,
        out_shape=jax.ShapeDtypeStruct((B, COUT), jnp.float32),
        compiler_params=pltpu.CompilerParams(
            dimension_semantics=("arbitrary",),
        ),
    )(meta, x_sorted, W, bias2d)


def kernel(in_feats, in_coords, W, bias):
    x_sorted, pos, meta = _sc_router(in_feats, in_coords)
    out_sorted = _gmm(x_sorted, W, bias.reshape(1, COUT), meta)
    return _sc_row_gather(out_sorted, pos)


# speedup vs baseline: 1.0919x; 1.0636x over previous
import functools

import jax
import jax.numpy as jnp
from jax import lax
from jax.experimental import pallas as pl
from jax.experimental.pallas import tpu as pltpu
from jax.experimental.pallas import tpu_sc as plsc

N = 64
H = 8
CIN = 768
COUT = 768
B = 8192
LAYER_NUM = 5

BM = 256
MT = B // BM
G = MT + N

NV = B // 16


def _sc_router(in_feats, in_coords):
    info = plsc.get_sparse_core_info()
    NC, NS = info.num_cores, info.num_subcores
    NW = NC * NS
    TPW = B // NW
    VPW = TPW // 16

    mesh = plsc.VectorSubcoreMesh(core_axis_name="c", subcore_axis_name="s")

    @functools.partial(
        pl.kernel,
        mesh=mesh,
        out_type=(
            jax.ShapeDtypeStruct((B, CIN), jnp.float32),
            jax.ShapeDtypeStruct((B,), jnp.int32),
            jax.ShapeDtypeStruct((4, G), jnp.int32),
        ),
        scratch_types=[
            pltpu.VMEM((2 * B,), jnp.float32),
            pltpu.VMEM((B,), jnp.int32),
            pltpu.VMEM((TPW,), jnp.int32),
            pltpu.VMEM((64,), jnp.int32),
            pltpu.VMEM((64, CIN), jnp.float32),
            pltpu.VMEM((16,), jnp.int32),
            pltpu.VMEM((16,), jnp.int32),
            pltpu.VMEM((N,), jnp.int32),
            pltpu.VMEM((N,), jnp.int32),
            pltpu.VMEM((N,), jnp.int32),
            pltpu.VMEM((N,), jnp.int32),
            pltpu.VMEM((N,), jnp.int32),
            pltpu.VMEM((N,), jnp.int32),
            pltpu.VMEM((G,), jnp.int32),
            pltpu.VMEM((G,), jnp.int32),
            pltpu.VMEM((G,), jnp.int32),
            pltpu.VMEM((G,), jnp.int32),
            pltpu.SemaphoreType.DMA,
        ],
        compiler_params=pltpu.CompilerParams(needs_layout_passes=False),
    )
    def k(feats_hbm, coords_hbm, xs_hbm, pos_hbm, meta_hbm,
          coords_v, tiles_v, pos_v, idx64_v, rows_v, skv, tmp16,
          cnt_all, cnt_pre, base_run, starts_v, blocks_v, u_v,
          grp_v, mt_v, lo_v, hi_v, sem):
        wid = lax.axis_index("s") * NC + lax.axis_index("c")
        iota = jnp.arange(16, dtype=jnp.int32)

        pltpu.sync_copy(coords_hbm, coords_v)

        def f2i_floor(a):
            t = a.astype(jnp.int32)
            return t - (t.astype(jnp.float32) > a).astype(jnp.int32)

        zeros16 = jnp.zeros((16,), jnp.int32)
        ones16 = jnp.ones((16,), jnp.int32)
        for c in range(N // 16):
            cnt_all[pl.ds(c * 16, 16)] = zeros16
            cnt_pre[pl.ds(c * 16, 16)] = zeros16

        def vreg_stats(keys):
            sk, sl = plsc.sort_key_val(keys, iota)
            skv[...] = sk
            prv = plsc.load_gather(skv, [jnp.maximum(iota - 1, 0)])
            nxt = plsc.load_gather(skv, [jnp.minimum(iota + 1, 15)])
            change = (iota == 0) | (sk != prv)
            first = plsc.cummax(jnp.where(change, iota, 0))
            rank = iota - first
            is_last = (iota == 15) | (sk != nxt)
            return sk, sl, rank, is_last

        myv0 = wid * VPW

        def hist_body(kv, _):
            rows2 = (kv * 16 + iota) * 2
            cx = plsc.load_gather(coords_v, [rows2])
            cy = plsc.load_gather(coords_v, [rows2 + 1])
            tx = f2i_floor(cx * 16.0 + 0.5) & (H - 1)
            ty = f2i_floor(cy * 16.0 + 0.5) & (H - 1)
            keys = tx * H + ty
            tiles_v[pl.ds(kv * 16, 16)] = keys
            plsc.addupdate_scatter(cnt_all, [keys], ones16)
            plsc.addupdate_scatter(
                cnt_pre, [keys], ones16,
                mask=jnp.broadcast_to(kv < myv0, (16,)),
            )
            return 0

        lax.fori_loop(0, NV, hist_body, 0)

        carry = jnp.int32(0)
        for c in range(N // 16):
            tot = cnt_all[pl.ds(c * 16, 16)]
            start = plsc.cumsum(tot) - tot + carry
            carry = carry + jnp.sum(tot)
            starts_v[pl.ds(c * 16, 16)] = start
            base_run[pl.ds(c * 16, 16)] = start + cnt_pre[pl.ds(c * 16, 16)]

        def pos_body(kv, _):
            keys = tiles_v[pl.ds((myv0 + kv) * 16, 16)]
            sk, sl, rank, is_last = vreg_stats(keys)
            basek = plsc.load_gather(base_run, [sk])
            plsc.store_scatter(tmp16, [sl], basek + rank)
            pos_v[pl.ds(kv * 16, 16)] = tmp16[...]
            plsc.addupdate_scatter(base_run, [sk], rank + 1, mask=is_last)
            return 0

        lax.fori_loop(0, VPW, pos_body, 0)
        pltpu.sync_copy(pos_v, pos_hbm.at[pl.ds(wid * TPW, TPW)])

        for c in range(TPW // 64):
            b0 = wid * TPW + c * 64
            pltpu.sync_copy(feats_hbm.at[pl.ds(b0, 64)], rows_v)
            for q in range(4):
                idx64_v[pl.ds(q * 16, 16)] = pos_v[pl.ds(c * 64 + q * 16, 16)]
            pltpu.async_copy(rows_v, xs_hbm.at[idx64_v], sem).wait()

        @pl.when(wid == 0)
        def _():
            carry2 = jnp.int32(0)
            for c in range(N // 16):
                sz = cnt_all[pl.ds(c * 16, 16)]
                st = starts_v[pl.ds(c * 16, 16)]
                en = st + sz
                blk = jnp.where(
                    sz > 0,
                    ((en - 1) >> 8) - (st >> 8) + 1,
                    0,
                )
                blocks_v[pl.ds(c * 16, 16)] = blk
                u_v[pl.ds(c * 16, 16)] = plsc.cumsum(blk) + carry2
                carry2 = carry2 + jnp.sum(blk)

            total = plsc.load_gather(u_v, [jnp.full((16,), N - 1, jnp.int32)])

            def meta_body(jv, _):
                j = jv * 16 + iota
                g0 = jnp.zeros((16,), jnp.int32)

                def count_body(t, g):
                    ut = plsc.load_gather(
                        u_v, [jnp.full((16,), t, jnp.int32)]
                    )
                    return g + (ut <= j).astype(jnp.int32)

                g = lax.fori_loop(0, N, count_body, g0)
                gc = jnp.minimum(g, N - 1)
                u_gc = plsc.load_gather(u_v, [gc])
                blk_gc = plsc.load_gather(blocks_v, [gc])
                st_gc = plsc.load_gather(starts_v, [gc])
                sz_gc = plsc.load_gather(cnt_all, [gc])
                en_gc = st_gc + sz_gc
                valid = j < total
                k_in = j - (u_gc - blk_gc)
                mtj = jnp.where(valid, (st_gc >> 8) + k_in, MT - 1)
                loj = jnp.where(valid, jnp.maximum(st_gc, mtj * BM), B)
                hij = jnp.where(valid, jnp.minimum(en_gc, (mtj + 1) * BM), B)
                grpj = jnp.where(valid, gc, N - 1)
                grp_v[pl.ds(jv * 16, 16)] = grpj
                mt_v[pl.ds(jv * 16, 16)] = mtj
                lo_v[pl.ds(jv * 16, 16)] = loj
                hi_v[pl.ds(jv * 16, 16)] = hij
                return 0

            lax.fori_loop(0, G // 16, meta_body, 0)
            pltpu.sync_copy(grp_v, meta_hbm.at[0])
            pltpu.sync_copy(mt_v, meta_hbm.at[1])
            pltpu.sync_copy(lo_v, meta_hbm.at[2])
            pltpu.sync_copy(hi_v, meta_hbm.at[3])

    return k(in_feats, in_coords.reshape(2 * B))


def _sc_row_gather(table, idx):
    R, D = table.shape
    info = plsc.get_sparse_core_info()
    NC, NS = info.num_cores, info.num_subcores
    NW = NC * NS
    rows_per_w = R // NW
    CH = 128
    n_chunks = rows_per_w // CH

    mesh = plsc.VectorSubcoreMesh(core_axis_name="c", subcore_axis_name="s")

    @functools.partial(
        pl.kernel,
        mesh=mesh,
        out_type=jax.ShapeDtypeStruct((R, D), table.dtype),
        scratch_types=[
            pltpu.VMEM((CH,), jnp.int32),
            pltpu.VMEM((CH, D), table.dtype),
            pltpu.SemaphoreType.DMA,
        ],
    )
    def k(table_hbm, idx_hbm, out_hbm, idx_v, rows_v, sem):
        wid = lax.axis_index("s") * NC + lax.axis_index("c")
        for c in range(n_chunks):
            base = wid * rows_per_w + c * CH
            pltpu.sync_copy(idx_hbm.at[pl.ds(base, CH)], idx_v)
            pltpu.async_copy(table_hbm.at[idx_v], rows_v, sem).wait()
            pltpu.sync_copy(rows_v, out_hbm.at[pl.ds(base, CH)])

    return k(table, idx)


def _gmm_body(meta_s, x_ref, w_ref, b_ref, o_ref):
    j = pl.program_id(0)
    mt = meta_s[1, j]
    lo = meta_s[2, j]
    hi = meta_s[3, j]
    rows = mt * BM + lax.broadcasted_iota(jnp.int32, (BM, 1), 0)
    mask = (rows >= lo) & (rows < hi)

    x = x_ref[...].astype(jnp.bfloat16)
    w = w_ref[0].astype(jnp.bfloat16)
    part = lax.dot_general(
        x, w, (((1,), (1,)), ((), ())), preferred_element_type=jnp.float32
    )
    part = jnp.where(mask, part, 0.0)

    prev_mt = meta_s[1, jnp.maximum(j - 1, 0)]
    next_mt = meta_s[1, jnp.minimum(j + 1, G - 1)]
    is_first = (j == 0) | (mt != prev_mt)
    is_last = (j == G - 1) | (mt != next_mt)

    prev = jnp.where(is_first, jnp.zeros_like(part), o_ref[...])
    acc = prev + part
    final = acc + b_ref[...]
    final = jnp.where(final >= 0, final, 0.2 * final)
    o_ref[...] = jnp.where(is_last, final, acc)


def _gmm(x_sorted, W, bias2d, meta):
    grid_spec = pltpu.PrefetchScalarGridSpec(
        num_scalar_prefetch=1,
        grid=(G,),
        in_specs=[
            pl.BlockSpec((BM, CIN), lambda j, m: (m[1, j], 0)),
            pl.BlockSpec((1, COUT, CIN), lambda j, m: (m[0, j], 0, 0)),
            pl.BlockSpec((1, COUT), lambda j, m: (0, 0)),
        ],
        out_specs=pl.BlockSpec((BM, COUT), lambda j, m: (m[1, j], 0)),
    )
    return pl.pallas_call(
        _gmm_body,
        grid_spec=grid_spec,
        out_shape=jax.ShapeDtypeStruct((B, COUT), jnp.float32),
        compiler_params=pltpu.CompilerParams(
            dimension_semantics=("arbitrary",),
        ),
    )(meta, x_sorted, W, bias2d)


def kernel(in_feats, in_coords, W, bias):
    x_sorted, pos, meta = _sc_router(in_feats, in_coords)
    out_sorted = _gmm(x_sorted, W, bias.reshape(1, COUT), meta)
    return _sc_row_gather(out_sorted, pos)


# speedup vs baseline: 1.0940x; 1.0019x over previous
import functools

import jax
import jax.numpy as jnp
from jax import lax
from jax.experimental import pallas as pl
from jax.experimental.pallas import tpu as pltpu
from jax.experimental.pallas import tpu_sc as plsc

N = 64
H = 8
CIN = 768
COUT = 768
B = 8192
LAYER_NUM = 5

BM = 256
MT = B // BM
G = MT + N

NV = B // 16


def _sc_router(in_feats, in_coords):
    info = plsc.get_sparse_core_info()
    NC, NS = info.num_cores, info.num_subcores
    NW = NC * NS
    TPW = B // NW
    VPW = TPW // 16

    mesh = plsc.VectorSubcoreMesh(core_axis_name="c", subcore_axis_name="s")

    @functools.partial(
        pl.kernel,
        mesh=mesh,
        out_type=(
            jax.ShapeDtypeStruct((B, CIN), jnp.float32),
            jax.ShapeDtypeStruct((B,), jnp.int32),
            jax.ShapeDtypeStruct((4, G), jnp.int32),
        ),
        scratch_types=[
            pltpu.VMEM((2 * B,), jnp.float32),
            pltpu.VMEM((B,), jnp.int32),
            pltpu.VMEM((TPW,), jnp.int32),
            pltpu.VMEM((64,), jnp.int32),
            pltpu.VMEM((64, CIN), jnp.float32),
            pltpu.VMEM((16,), jnp.int32),
            pltpu.VMEM((16,), jnp.int32),
            pltpu.VMEM((N,), jnp.int32),
            pltpu.VMEM((N,), jnp.int32),
            pltpu.VMEM((N,), jnp.int32),
            pltpu.VMEM((N,), jnp.int32),
            pltpu.VMEM((N,), jnp.int32),
            pltpu.VMEM((N,), jnp.int32),
            pltpu.VMEM((G,), jnp.int32),
            pltpu.VMEM((G,), jnp.int32),
            pltpu.VMEM((G,), jnp.int32),
            pltpu.VMEM((G,), jnp.int32),
            pltpu.SemaphoreType.DMA,
        ],
        compiler_params=pltpu.CompilerParams(needs_layout_passes=False),
    )
    def k(feats_hbm, coords_hbm, xs_hbm, pos_hbm, meta_hbm,
          coords_v, tiles_v, pos_v, idx64_v, rows_v, skv, tmp16,
          cnt_all, cnt_pre, base_run, starts_v, blocks_v, u_v,
          grp_v, mt_v, lo_v, hi_v, sem):
        wid = lax.axis_index("s") * NC + lax.axis_index("c")
        iota = jnp.arange(16, dtype=jnp.int32)

        pltpu.sync_copy(coords_hbm, coords_v)

        def f2i_floor(a):
            t = a.astype(jnp.int32)
            return t - (t.astype(jnp.float32) > a).astype(jnp.int32)

        zeros16 = jnp.zeros((16,), jnp.int32)
        ones16 = jnp.ones((16,), jnp.int32)
        for c in range(N // 16):
            cnt_all[pl.ds(c * 16, 16)] = zeros16
            cnt_pre[pl.ds(c * 16, 16)] = zeros16

        def vreg_stats(keys):
            sk, sl = plsc.sort_key_val(keys, iota)
            skv[...] = sk
            prv = plsc.load_gather(skv, [jnp.maximum(iota - 1, 0)])
            nxt = plsc.load_gather(skv, [jnp.minimum(iota + 1, 15)])
            change = (iota == 0) | (sk != prv)
            first = plsc.cummax(jnp.where(change, iota, 0))
            rank = iota - first
            is_last = (iota == 15) | (sk != nxt)
            return sk, sl, rank, is_last

        myv0 = wid * VPW

        def hist_body(kv, _):
            rows2 = (kv * 16 + iota) * 2
            cx = plsc.load_gather(coords_v, [rows2])
            cy = plsc.load_gather(coords_v, [rows2 + 1])
            tx = f2i_floor(cx * 16.0 + 0.5) & (H - 1)
            ty = f2i_floor(cy * 16.0 + 0.5) & (H - 1)
            keys = tx * H + ty
            tiles_v[pl.ds(kv * 16, 16)] = keys
            plsc.addupdate_scatter(cnt_all, [keys], ones16)
            plsc.addupdate_scatter(
                cnt_pre, [keys], ones16,
                mask=jnp.broadcast_to(kv < myv0, (16,)),
            )
            return 0

        lax.fori_loop(0, NV, hist_body, 0)

        carry = jnp.int32(0)
        for c in range(N // 16):
            tot = cnt_all[pl.ds(c * 16, 16)]
            start = plsc.cumsum(tot) - tot + carry
            carry = carry + jnp.sum(tot)
            starts_v[pl.ds(c * 16, 16)] = start
            base_run[pl.ds(c * 16, 16)] = start + cnt_pre[pl.ds(c * 16, 16)]

        def pos_body(kv, _):
            keys = tiles_v[pl.ds((myv0 + kv) * 16, 16)]
            sk, sl, rank, is_last = vreg_stats(keys)
            basek = plsc.load_gather(base_run, [sk])
            plsc.store_scatter(tmp16, [sl], basek + rank)
            pos_v[pl.ds(kv * 16, 16)] = tmp16[...]
            plsc.addupdate_scatter(base_run, [sk], rank + 1, mask=is_last)
            return 0

        lax.fori_loop(0, VPW, pos_body, 0)
        pltpu.sync_copy(pos_v, pos_hbm.at[pl.ds(wid * TPW, TPW)])

        for c in range(TPW // 64):
            b0 = wid * TPW + c * 64
            pltpu.sync_copy(feats_hbm.at[pl.ds(b0, 64)], rows_v)
            for q in range(4):
                idx64_v[pl.ds(q * 16, 16)] = pos_v[pl.ds(c * 64 + q * 16, 16)]
            pltpu.async_copy(rows_v, xs_hbm.at[idx64_v], sem).wait()

        @pl.when(wid == 0)
        def _():
            carry2 = jnp.int32(0)
            for c in range(N // 16):
                sz = cnt_all[pl.ds(c * 16, 16)]
                st = starts_v[pl.ds(c * 16, 16)]
                en = st + sz
                blk = jnp.where(
                    sz > 0,
                    ((en - 1) >> 8) - (st >> 8) + 1,
                    0,
                )
                blocks_v[pl.ds(c * 16, 16)] = blk
                u_v[pl.ds(c * 16, 16)] = plsc.cumsum(blk) + carry2
                carry2 = carry2 + jnp.sum(blk)

            total = plsc.load_gather(u_v, [jnp.full((16,), N - 1, jnp.int32)])

            def meta_body(jv, _):
                j = jv * 16 + iota
                g0 = jnp.zeros((16,), jnp.int32)

                def count_body(t, g):
                    ut = plsc.load_gather(
                        u_v, [jnp.full((16,), t, jnp.int32)]
                    )
                    return g + (ut <= j).astype(jnp.int32)

                g = lax.fori_loop(0, N, count_body, g0)
                gc = jnp.minimum(g, N - 1)
                u_gc = plsc.load_gather(u_v, [gc])
                blk_gc = plsc.load_gather(blocks_v, [gc])
                st_gc = plsc.load_gather(starts_v, [gc])
                sz_gc = plsc.load_gather(cnt_all, [gc])
                en_gc = st_gc + sz_gc
                valid = j < total
                k_in = j - (u_gc - blk_gc)
                mtj = jnp.where(valid, (st_gc >> 8) + k_in, MT - 1)
                loj = jnp.where(valid, jnp.maximum(st_gc, mtj * BM), B)
                hij = jnp.where(valid, jnp.minimum(en_gc, (mtj + 1) * BM), B)
                grpj = jnp.where(valid, gc, N - 1)
                grp_v[pl.ds(jv * 16, 16)] = grpj
                mt_v[pl.ds(jv * 16, 16)] = mtj
                lo_v[pl.ds(jv * 16, 16)] = loj
                hi_v[pl.ds(jv * 16, 16)] = hij
                return 0

            lax.fori_loop(0, G // 16, meta_body, 0)
            pltpu.sync_copy(grp_v, meta_hbm.at[0])
            pltpu.sync_copy(mt_v, meta_hbm.at[1])
            pltpu.sync_copy(lo_v, meta_hbm.at[2])
            pltpu.sync_copy(hi_v, meta_hbm.at[3])

    return k(in_feats, in_coords.reshape(2 * B))


def _sc_row_gather(table, idx):
    R, D = table.shape
    info = plsc.get_sparse_core_info()
    NC, NS = info.num_cores, info.num_subcores
    NW = NC * NS
    rows_per_w = R // NW
    CH = 64
    n_chunks = rows_per_w // CH

    mesh = plsc.VectorSubcoreMesh(core_axis_name="c", subcore_axis_name="s")

    @functools.partial(
        pl.kernel,
        mesh=mesh,
        out_type=jax.ShapeDtypeStruct((R, D), table.dtype),
        scratch_types=[
            pltpu.VMEM((CH,), jnp.int32),
            pltpu.VMEM((CH,), jnp.int32),
            pltpu.VMEM((CH, D), table.dtype),
            pltpu.VMEM((CH, D), table.dtype),
            pltpu.SemaphoreType.DMA,
            pltpu.SemaphoreType.DMA,
            pltpu.SemaphoreType.DMA,
            pltpu.SemaphoreType.DMA,
        ],
    )
    def k(table_hbm, idx_hbm, out_hbm, i0, i1, r0, r1, g0, g1, w0, w1):
        wid = lax.axis_index("s") * NC + lax.axis_index("c")
        idxs, rows, gs, ws = [i0, i1], [r0, r1], [g0, g1], [w0, w1]
        gh, wh = [None, None], [None, None]
        for c in range(n_chunks):
            p = c % 2
            if wh[p] is not None:
                wh[p].wait()
            base = wid * rows_per_w + c * CH
            pltpu.sync_copy(idx_hbm.at[pl.ds(base, CH)], idxs[p])
            gh[p] = pltpu.async_copy(table_hbm.at[idxs[p]], rows[p], gs[p])
            if c > 0:
                q = (c - 1) % 2
                gh[q].wait()
                b_prev = wid * rows_per_w + (c - 1) * CH
                wh[q] = pltpu.async_copy(
                    rows[q], out_hbm.at[pl.ds(b_prev, CH)], ws[q]
                )
        p = (n_chunks - 1) % 2
        gh[p].wait()
        b_last = wid * rows_per_w + (n_chunks - 1) * CH
        wh[p] = pltpu.async_copy(rows[p], out_hbm.at[pl.ds(b_last, CH)], ws[p])
        wh[1 - p].wait()
        wh[p].wait()

    return k(table, idx)


def _gmm_body(meta_s, x_ref, w_ref, b_ref, o_ref):
    j = pl.program_id(0)
    mt = meta_s[1, j]
    lo = meta_s[2, j]
    hi = meta_s[3, j]
    rows = mt * BM + lax.broadcasted_iota(jnp.int32, (BM, 1), 0)
    mask = (rows >= lo) & (rows < hi)

    x = x_ref[...].astype(jnp.bfloat16)
    w = w_ref[0].astype(jnp.bfloat16)
    part = lax.dot_general(
        x, w, (((1,), (1,)), ((), ())), preferred_element_type=jnp.float32
    )
    part = jnp.where(mask, part, 0.0)

    prev_mt = meta_s[1, jnp.maximum(j - 1, 0)]
    next_mt = meta_s[1, jnp.minimum(j + 1, G - 1)]
    is_first = (j == 0) | (mt != prev_mt)
    is_last = (j == G - 1) | (mt != next_mt)

    prev = jnp.where(is_first, jnp.zeros_like(part), o_ref[...])
    acc = prev + part
    final = acc + b_ref[...]
    final = jnp.where(final >= 0, final, 0.2 * final)
    o_ref[...] = jnp.where(is_last, final, acc)


def _gmm(x_sorted, W, bias2d, meta):
    grid_spec = pltpu.PrefetchScalarGridSpec(
        num_scalar_prefetch=1,
        grid=(G,),
        in_specs=[
            pl.BlockSpec((BM, CIN), lambda j, m: (m[1, j], 0)),
            pl.BlockSpec((1, COUT, CIN), lambda j, m: (m[0, j], 0, 0)),
            pl.BlockSpec((1, COUT), lambda j, m: (0, 0)),
        ],
        out_specs=pl.BlockSpec((BM, COUT), lambda j, m: (m[1, j], 0)),
    )
    return pl.pallas_call(
        _gmm_body,
        grid_spec=grid_spec,
        out_shape=jax.ShapeDtypeStruct((B, COUT), jnp.float32),
        compiler_params=pltpu.CompilerParams(
            dimension_semantics=("arbitrary",),
        ),
    )(meta, x_sorted, W, bias2d)


def kernel(in_feats, in_coords, W, bias):
    x_sorted, pos, meta = _sc_router(in_feats, in_coords)
    out_sorted = _gmm(x_sorted, W, bias.reshape(1, COUT), meta)
    return _sc_row_gather(out_sorted, pos)


# speedup vs baseline: 1.1007x; 1.0061x over previous
import functools

import jax
import jax.numpy as jnp
from jax import lax
from jax.experimental import pallas as pl
from jax.experimental.pallas import tpu as pltpu
from jax.experimental.pallas import tpu_sc as plsc

N = 64
H = 8
CIN = 768
COUT = 768
B = 8192
LAYER_NUM = 5

BM = 256
MT = B // BM
G = MT + N

NV = B // 16


def _sc_router(in_feats, in_coords):
    info = plsc.get_sparse_core_info()
    NC, NS = info.num_cores, info.num_subcores
    NW = NC * NS
    TPW = B // NW
    VPW = TPW // 16

    mesh = plsc.VectorSubcoreMesh(core_axis_name="c", subcore_axis_name="s")

    @functools.partial(
        pl.kernel,
        mesh=mesh,
        out_type=(
            jax.ShapeDtypeStruct((B, CIN), jnp.float32),
            jax.ShapeDtypeStruct((B,), jnp.int32),
            jax.ShapeDtypeStruct((4, G), jnp.int32),
        ),
        scratch_types=[
            pltpu.VMEM((2 * B,), jnp.float32),
            pltpu.VMEM((B,), jnp.int32),
            pltpu.VMEM((TPW,), jnp.int32),
            pltpu.VMEM((64,), jnp.int32),
            pltpu.VMEM((64,), jnp.int32),
            pltpu.VMEM((64, CIN), jnp.float32),
            pltpu.VMEM((64, CIN), jnp.float32),
            pltpu.VMEM((16,), jnp.int32),
            pltpu.VMEM((16,), jnp.int32),
            pltpu.VMEM((N,), jnp.int32),
            pltpu.VMEM((N,), jnp.int32),
            pltpu.VMEM((N,), jnp.int32),
            pltpu.VMEM((N,), jnp.int32),
            pltpu.VMEM((N,), jnp.int32),
            pltpu.VMEM((N,), jnp.int32),
            pltpu.VMEM((G,), jnp.int32),
            pltpu.VMEM((G,), jnp.int32),
            pltpu.VMEM((G,), jnp.int32),
            pltpu.VMEM((G,), jnp.int32),
            pltpu.SemaphoreType.DMA,
            pltpu.SemaphoreType.DMA,
            pltpu.SemaphoreType.DMA,
            pltpu.SemaphoreType.DMA,
        ],
        compiler_params=pltpu.CompilerParams(needs_layout_passes=False),
    )
    def k(feats_hbm, coords_hbm, xs_hbm, pos_hbm, meta_hbm,
          coords_v, tiles_v, pos_v, idx64_v, idx64_b, rows_v, rows_b,
          skv, tmp16,
          cnt_all, cnt_pre, base_run, starts_v, blocks_v, u_v,
          grp_v, mt_v, lo_v, hi_v, lsem0, lsem1, ssem0, ssem1):
        wid = lax.axis_index("s") * NC + lax.axis_index("c")
        iota = jnp.arange(16, dtype=jnp.int32)

        pltpu.sync_copy(coords_hbm, coords_v)

        def f2i_floor(a):
            t = a.astype(jnp.int32)
            return t - (t.astype(jnp.float32) > a).astype(jnp.int32)

        zeros16 = jnp.zeros((16,), jnp.int32)
        ones16 = jnp.ones((16,), jnp.int32)
        for c in range(N // 16):
            cnt_all[pl.ds(c * 16, 16)] = zeros16
            cnt_pre[pl.ds(c * 16, 16)] = zeros16

        def vreg_stats(keys):
            sk, sl = plsc.sort_key_val(keys, iota)
            skv[...] = sk
            prv = plsc.load_gather(skv, [jnp.maximum(iota - 1, 0)])
            nxt = plsc.load_gather(skv, [jnp.minimum(iota + 1, 15)])
            change = (iota == 0) | (sk != prv)
            first = plsc.cummax(jnp.where(change, iota, 0))
            rank = iota - first
            is_last = (iota == 15) | (sk != nxt)
            return sk, sl, rank, is_last

        myv0 = wid * VPW

        def hist_body(kv, _):
            rows2 = (kv * 16 + iota) * 2
            cx = plsc.load_gather(coords_v, [rows2])
            cy = plsc.load_gather(coords_v, [rows2 + 1])
            tx = f2i_floor(cx * 16.0 + 0.5) & (H - 1)
            ty = f2i_floor(cy * 16.0 + 0.5) & (H - 1)
            keys = tx * H + ty
            tiles_v[pl.ds(kv * 16, 16)] = keys
            plsc.addupdate_scatter(cnt_all, [keys], ones16)
            plsc.addupdate_scatter(
                cnt_pre, [keys], ones16,
                mask=jnp.broadcast_to(kv < myv0, (16,)),
            )
            return 0

        lax.fori_loop(0, NV, hist_body, 0)

        carry = jnp.int32(0)
        for c in range(N // 16):
            tot = cnt_all[pl.ds(c * 16, 16)]
            start = plsc.cumsum(tot) - tot + carry
            carry = carry + jnp.sum(tot)
            starts_v[pl.ds(c * 16, 16)] = start
            base_run[pl.ds(c * 16, 16)] = start + cnt_pre[pl.ds(c * 16, 16)]

        def pos_body(kv, _):
            keys = tiles_v[pl.ds((myv0 + kv) * 16, 16)]
            sk, sl, rank, is_last = vreg_stats(keys)
            basek = plsc.load_gather(base_run, [sk])
            plsc.store_scatter(tmp16, [sl], basek + rank)
            pos_v[pl.ds(kv * 16, 16)] = tmp16[...]
            plsc.addupdate_scatter(base_run, [sk], rank + 1, mask=is_last)
            return 0

        lax.fori_loop(0, VPW, pos_body, 0)
        pltpu.sync_copy(pos_v, pos_hbm.at[pl.ds(wid * TPW, TPW)])

        rbufs, ibufs = [rows_v, rows_b], [idx64_v, idx64_b]
        lsems, ssems = [lsem0, lsem1], [ssem0, ssem1]
        lh, sh = [None, None], [None, None]
        n_ch = TPW // 64
        for c in range(n_ch):
            p = c % 2
            if sh[p] is not None:
                sh[p].wait()
            b0 = wid * TPW + c * 64
            lh[p] = pltpu.async_copy(feats_hbm.at[pl.ds(b0, 64)], rbufs[p],
                                     lsems[p])
            for q in range(4):
                ibufs[p][pl.ds(q * 16, 16)] = pos_v[pl.ds(c * 64 + q * 16, 16)]
            if c > 0:
                qq = (c - 1) % 2
                lh[qq].wait()
                sh[qq] = pltpu.async_copy(rbufs[qq], xs_hbm.at[ibufs[qq]],
                                          ssems[qq])
        p = (n_ch - 1) % 2
        lh[p].wait()
        sh[p] = pltpu.async_copy(rbufs[p], xs_hbm.at[ibufs[p]], ssems[p])
        sh[1 - p].wait()
        sh[p].wait()

        @pl.when(wid == 0)
        def _():
            carry2 = jnp.int32(0)
            for c in range(N // 16):
                sz = cnt_all[pl.ds(c * 16, 16)]
                st = starts_v[pl.ds(c * 16, 16)]
                en = st + sz
                blk = jnp.where(
                    sz > 0,
                    ((en - 1) >> 8) - (st >> 8) + 1,
                    0,
                )
                blocks_v[pl.ds(c * 16, 16)] = blk
                u_v[pl.ds(c * 16, 16)] = plsc.cumsum(blk) + carry2
                carry2 = carry2 + jnp.sum(blk)

            total = plsc.load_gather(u_v, [jnp.full((16,), N - 1, jnp.int32)])

            def meta_body(jv, _):
                j = jv * 16 + iota
                g0 = jnp.zeros((16,), jnp.int32)

                def count_body(t, g):
                    ut = plsc.load_gather(
                        u_v, [jnp.full((16,), t, jnp.int32)]
                    )
                    return g + (ut <= j).astype(jnp.int32)

                g = lax.fori_loop(0, N, count_body, g0)
                gc = jnp.minimum(g, N - 1)
                u_gc = plsc.load_gather(u_v, [gc])
                blk_gc = plsc.load_gather(blocks_v, [gc])
                st_gc = plsc.load_gather(starts_v, [gc])
                sz_gc = plsc.load_gather(cnt_all, [gc])
                en_gc = st_gc + sz_gc
                valid = j < total
                k_in = j - (u_gc - blk_gc)
                mtj = jnp.where(valid, (st_gc >> 8) + k_in, MT - 1)
                loj = jnp.where(valid, jnp.maximum(st_gc, mtj * BM), B)
                hij = jnp.where(valid, jnp.minimum(en_gc, (mtj + 1) * BM), B)
                grpj = jnp.where(valid, gc, N - 1)
                grp_v[pl.ds(jv * 16, 16)] = grpj
                mt_v[pl.ds(jv * 16, 16)] = mtj
                lo_v[pl.ds(jv * 16, 16)] = loj
                hi_v[pl.ds(jv * 16, 16)] = hij
                return 0

            lax.fori_loop(0, G // 16, meta_body, 0)
            pltpu.sync_copy(grp_v, meta_hbm.at[0])
            pltpu.sync_copy(mt_v, meta_hbm.at[1])
            pltpu.sync_copy(lo_v, meta_hbm.at[2])
            pltpu.sync_copy(hi_v, meta_hbm.at[3])

    return k(in_feats, in_coords.reshape(2 * B))


def _sc_row_gather(table, idx):
    R, D = table.shape
    info = plsc.get_sparse_core_info()
    NC, NS = info.num_cores, info.num_subcores
    NW = NC * NS
    rows_per_w = R // NW
    CH = 64
    n_chunks = rows_per_w // CH

    mesh = plsc.VectorSubcoreMesh(core_axis_name="c", subcore_axis_name="s")

    @functools.partial(
        pl.kernel,
        mesh=mesh,
        out_type=jax.ShapeDtypeStruct((R, D), table.dtype),
        scratch_types=[
            pltpu.VMEM((CH,), jnp.int32),
            pltpu.VMEM((CH,), jnp.int32),
            pltpu.VMEM((CH, D), table.dtype),
            pltpu.VMEM((CH, D), table.dtype),
            pltpu.SemaphoreType.DMA,
            pltpu.SemaphoreType.DMA,
            pltpu.SemaphoreType.DMA,
            pltpu.SemaphoreType.DMA,
        ],
    )
    def k(table_hbm, idx_hbm, out_hbm, i0, i1, r0, r1, g0, g1, w0, w1):
        wid = lax.axis_index("s") * NC + lax.axis_index("c")
        idxs, rows, gs, ws = [i0, i1], [r0, r1], [g0, g1], [w0, w1]
        gh, wh = [None, None], [None, None]
        for c in range(n_chunks):
            p = c % 2
            if wh[p] is not None:
                wh[p].wait()
            base = wid * rows_per_w + c * CH
            pltpu.sync_copy(idx_hbm.at[pl.ds(base, CH)], idxs[p])
            gh[p] = pltpu.async_copy(table_hbm.at[idxs[p]], rows[p], gs[p])
            if c > 0:
                q = (c - 1) % 2
                gh[q].wait()
                b_prev = wid * rows_per_w + (c - 1) * CH
                wh[q] = pltpu.async_copy(
                    rows[q], out_hbm.at[pl.ds(b_prev, CH)], ws[q]
                )
        p = (n_chunks - 1) % 2
        gh[p].wait()
        b_last = wid * rows_per_w + (n_chunks - 1) * CH
        wh[p] = pltpu.async_copy(rows[p], out_hbm.at[pl.ds(b_last, CH)], ws[p])
        wh[1 - p].wait()
        wh[p].wait()

    return k(table, idx)


def _gmm_body(meta_s, x_ref, w_ref, b_ref, o_ref):
    j = pl.program_id(0)
    mt = meta_s[1, j]
    lo = meta_s[2, j]
    hi = meta_s[3, j]
    rows = mt * BM + lax.broadcasted_iota(jnp.int32, (BM, 1), 0)
    mask = (rows >= lo) & (rows < hi)

    x = x_ref[...].astype(jnp.bfloat16)
    w = w_ref[0].astype(jnp.bfloat16)
    part = lax.dot_general(
        x, w, (((1,), (1,)), ((), ())), preferred_element_type=jnp.float32
    )
    part = jnp.where(mask, part, 0.0)

    prev_mt = meta_s[1, jnp.maximum(j - 1, 0)]
    next_mt = meta_s[1, jnp.minimum(j + 1, G - 1)]
    is_first = (j == 0) | (mt != prev_mt)
    is_last = (j == G - 1) | (mt != next_mt)

    prev = jnp.where(is_first, jnp.zeros_like(part), o_ref[...])
    acc = prev + part
    final = acc + b_ref[...]
    final = jnp.where(final >= 0, final, 0.2 * final)
    o_ref[...] = jnp.where(is_last, final, acc)


def _gmm(x_sorted, W, bias2d, meta):
    grid_spec = pltpu.PrefetchScalarGridSpec(
        num_scalar_prefetch=1,
        grid=(G,),
        in_specs=[
            pl.BlockSpec((BM, CIN), lambda j, m: (m[1, j], 0)),
            pl.BlockSpec((1, COUT, CIN), lambda j, m: (m[0, j], 0, 0)),
            pl.BlockSpec((1, COUT), lambda j, m: (0, 0)),
        ],
        out_specs=pl.BlockSpec((BM, COUT), lambda j, m: (m[1, j], 0)),
    )
    return pl.pallas_call(
        _gmm_body,
        grid_spec=grid_spec,
        out_shape=jax.ShapeDtypeStruct((B, COUT), jnp.float32),
        compiler_params=pltpu.CompilerParams(
            dimension_semantics=("arbitrary",),
        ),
    )(meta, x_sorted, W, bias2d)


def kernel(in_feats, in_coords, W, bias):
    x_sorted, pos, meta = _sc_router(in_feats, in_coords)
    out_sorted = _gmm(x_sorted, W, bias.reshape(1, COUT), meta)
    return _sc_row_gather(out_sorted, pos)
